# baseline (speedup 1.0000x reference)
import functools

import jax
import jax.numpy as jnp
from jax import lax
from jax.experimental import pallas as pl
from jax.experimental.pallas import tpu as pltpu

H = 2048
E = 16
NF = 16384
FE = NF // E

HB = 512
BF = 256
KA = H // HB
KB = FE // BF
GRID = KA + KB


def _router_body(act_ref, rb_ref, router_ref, idx_ref, maxp_ref):
    x = act_ref[...] - rb_ref[...]
    logits = jnp.dot(x, router_ref[...],
                     preferred_element_type=jnp.float32)
    m = jnp.max(logits)
    s = jnp.sum(jnp.exp(logits - m))
    iota = lax.broadcasted_iota(jnp.int32, (1, E), 1)
    idx = jnp.min(jnp.where(logits == m, iota, E))
    idx_ref[0] = idx
    maxp_ref[0] = 1.0 / s


def _main_body(idx_ref, act_ref, eb_ref, maxp_ref, enc_ref, dec_ref, out_ref,
               f_ref):
    i = pl.program_id(0)

    @pl.when(i < KA)
    def _enc_phase():
        h0 = jnp.minimum(i, KA - 1) * HB
        xb = (act_ref[:, pl.ds(h0, HB)]
              - eb_ref[:, pl.ds(h0, HB)])
        pf = jnp.dot(xb, enc_ref[0],
                     preferred_element_type=jnp.float32)

        @pl.when(i == 0)
        def _():
            f_ref[...] = pf

        @pl.when(i > 0)
        def _():
            f_ref[...] += pf

    @pl.when(i >= KA)
    def _dec_phase():
        j = jnp.maximum(i - KA, 0)
        fb = jnp.maximum(f_ref[:, pl.ds(j * BF, BF)], 0.0)
        contrib = jnp.dot(fb, dec_ref[0],
                          preferred_element_type=jnp.float32)

        @pl.when(i == KA)
        def _():
            out_ref[...] = contrib

        @pl.when(i > KA)
        def _():
            out_ref[...] += contrib

        @pl.when(i == GRID - 1)
        def _():
            out_ref[...] = out_ref[...] * maxp_ref[0] + eb_ref[...]


def kernel(activations, enc, dec, expert_b, router_b, router):
    act2 = activations.reshape(1, H)
    rb2 = router_b.reshape(1, H)
    eb2 = expert_b.reshape(1, H)

    idx, maxp = pl.pallas_call(
        _router_body,
        out_shape=[
            jax.ShapeDtypeStruct((1,), jnp.int32),
            jax.ShapeDtypeStruct((1,), jnp.float32),
        ],
        in_specs=[
            pl.BlockSpec(memory_space=pltpu.VMEM),
            pl.BlockSpec(memory_space=pltpu.VMEM),
            pl.BlockSpec(memory_space=pltpu.VMEM),
        ],
        out_specs=[
            pl.BlockSpec(memory_space=pltpu.SMEM),
            pl.BlockSpec(memory_space=pltpu.SMEM),
        ],
    )(act2, rb2, router)

    out = pl.pallas_call(
        _main_body,
        grid_spec=pltpu.PrefetchScalarGridSpec(
            num_scalar_prefetch=1,
            grid=(GRID,),
            in_specs=[
                pl.BlockSpec((1, H), lambda i, idx_ref: (0, 0)),
                pl.BlockSpec((1, H), lambda i, idx_ref: (0, 0)),
                pl.BlockSpec(memory_space=pltpu.SMEM),
                pl.BlockSpec(
                    (1, HB, FE),
                    lambda i, idx_ref: (idx_ref[0],
                                        jnp.minimum(i, KA - 1), 0)),
                pl.BlockSpec(
                    (1, BF, H),
                    lambda i, idx_ref: (idx_ref[0],
                                        jnp.maximum(i - KA, 0), 0)),
            ],
            out_specs=pl.BlockSpec((1, H), lambda i, idx_ref: (0, 0)),
            scratch_shapes=[pltpu.VMEM((1, FE), jnp.float32)],
        ),
        out_shape=jax.ShapeDtypeStruct((1, H), jnp.float32),
    )(idx, act2, eb2, maxp, enc, dec)

    return out.reshape(H)


# speedup vs baseline: 1.1179x; 1.1179x over previous
import functools

import jax
import jax.numpy as jnp
from jax import lax
from jax.experimental import pallas as pl
from jax.experimental.pallas import tpu as pltpu

H = 2048
E = 16
NF = 16384
FE = NF // E

G = 4
NS = 2
W = FE // G
WS = W // NS


def _router_body(act_ref, rb_ref, router_ref, idx_ref, maxp_ref):
    x = act_ref[...] - rb_ref[...]
    logits = jnp.dot(x, router_ref[...],
                     preferred_element_type=jnp.float32)
    m = jnp.max(logits)
    s = jnp.sum(jnp.exp(logits - m))
    iota = lax.broadcasted_iota(jnp.int32, (1, E), 1)
    idx = jnp.min(jnp.where(logits == m, iota, E))
    idx_ref[0] = idx
    maxp_ref[0] = 1.0 / s


def _main_body(idx_ref, act_ref, eb_ref, maxp_ref, *refs):
    enc_refs = refs[:NS]
    dec_refs = refs[NS:2 * NS]
    out_ref = refs[2 * NS]
    i = pl.program_id(0)
    x = act_ref[...] - eb_ref[...]
    contrib = None
    for s in range(NS):
        f = jnp.dot(x, enc_refs[s][0],
                    preferred_element_type=jnp.float32)
        f = jnp.maximum(f, 0.0)
        c = jnp.dot(f, dec_refs[s][0],
                    preferred_element_type=jnp.float32)
        contrib = c if contrib is None else contrib + c

    @pl.when(i == 0)
    def _init():
        out_ref[...] = contrib

    @pl.when(i > 0)
    def _acc():
        out_ref[...] += contrib

    @pl.when(i == G - 1)
    def _fin():
        out_ref[...] = out_ref[...] * maxp_ref[0] + eb_ref[...]


def _enc_spec(s):
    return pl.BlockSpec((1, H, WS),
                        lambda i, idx_ref: (idx_ref[0], 0, i * NS + s))


def _dec_spec(s):
    return pl.BlockSpec((1, WS, H),
                        lambda i, idx_ref: (idx_ref[0], i * NS + s, 0))


def kernel(activations, enc, dec, expert_b, router_b, router):
    act2 = activations.reshape(1, H)
    rb2 = router_b.reshape(1, H)
    eb2 = expert_b.reshape(1, H)

    idx, maxp = pl.pallas_call(
        _router_body,
        out_shape=[
            jax.ShapeDtypeStruct((1,), jnp.int32),
            jax.ShapeDtypeStruct((1,), jnp.float32),
        ],
        in_specs=[
            pl.BlockSpec(memory_space=pltpu.VMEM),
            pl.BlockSpec(memory_space=pltpu.VMEM),
            pl.BlockSpec(memory_space=pltpu.VMEM),
        ],
        out_specs=[
            pl.BlockSpec(memory_space=pltpu.SMEM),
            pl.BlockSpec(memory_space=pltpu.SMEM),
        ],
    )(act2, rb2, router)

    out = pl.pallas_call(
        _main_body,
        grid_spec=pltpu.PrefetchScalarGridSpec(
            num_scalar_prefetch=1,
            grid=(G,),
            in_specs=[
                pl.BlockSpec((1, H), lambda i, idx_ref: (0, 0)),
                pl.BlockSpec((1, H), lambda i, idx_ref: (0, 0)),
                pl.BlockSpec(memory_space=pltpu.SMEM),
            ] + [_enc_spec(s) for s in range(NS)]
              + [_dec_spec(s) for s in range(NS)],
            out_specs=pl.BlockSpec((1, H), lambda i, idx_ref: (0, 0)),
        ),
        out_shape=jax.ShapeDtypeStruct((1, H), jnp.float32),
    )(idx, act2, eb2, maxp, *([enc] * NS), *([dec] * NS))

    return out.reshape(H)


# speedup vs baseline: 1.4015x; 1.2537x over previous
import jax
import jax.numpy as jnp
from jax import lax
from jax.experimental import pallas as pl
from jax.experimental.pallas import tpu as pltpu

H = 2048
E = 16
NF = 16384
FE = NF // E

CE = 4
CD = 4
HB = H // CE
WB = FE // CD


def _body(act_ref, eb_ref, rb_ref, router_ref, enc_hbm, dec_hbm, out_ref,
          enc_buf, dec_buf, enc_sems, dec_sems):
    xr = act_ref[...] - rb_ref[...]
    logits = jnp.dot(xr, router_ref[...],
                     preferred_element_type=jnp.float32)
    m = jnp.max(logits)
    maxp = 1.0 / jnp.sum(jnp.exp(logits - m))
    iota = lax.broadcasted_iota(jnp.int32, (1, E), 1)
    idx = jnp.min(jnp.where(logits == m, iota, E))

    enc_copies = [
        pltpu.make_async_copy(
            enc_hbm.at[idx, pl.ds(k * HB, HB), :],
            enc_buf.at[pl.ds(k * HB, HB), :],
            enc_sems.at[k],
        )
        for k in range(CE)
    ]
    dec_copies = [
        pltpu.make_async_copy(
            dec_hbm.at[idx, pl.ds(k * WB, WB), :],
            dec_buf.at[pl.ds(k * WB, WB), :],
            dec_sems.at[k],
        )
        for k in range(CD)
    ]
    for c in enc_copies:
        c.start()
    for c in dec_copies:
        c.start()

    x = act_ref[...] - eb_ref[...]
    f = None
    for k in range(CE):
        enc_copies[k].wait()
        pf = jnp.dot(x[:, k * HB:(k + 1) * HB],
                     enc_buf[k * HB:(k + 1) * HB, :],
                     preferred_element_type=jnp.float32)
        f = pf if f is None else f + pf
    f = jnp.maximum(f, 0.0)

    acc = None
    for k in range(CD):
        dec_copies[k].wait()
        c = jnp.dot(f[:, k * WB:(k + 1) * WB],
                    dec_buf[k * WB:(k + 1) * WB, :],
                    preferred_element_type=jnp.float32)
        acc = c if acc is None else acc + c

    out_ref[...] = acc * maxp + eb_ref[...]


def kernel(activations, enc, dec, expert_b, router_b, router):
    act2 = activations.reshape(1, H)
    rb2 = router_b.reshape(1, H)
    eb2 = expert_b.reshape(1, H)

    out = pl.pallas_call(
        _body,
        in_specs=[
            pl.BlockSpec(memory_space=pltpu.VMEM),
            pl.BlockSpec(memory_space=pltpu.VMEM),
            pl.BlockSpec(memory_space=pltpu.VMEM),
            pl.BlockSpec(memory_space=pltpu.VMEM),
            pl.BlockSpec(memory_space=pl.ANY),
            pl.BlockSpec(memory_space=pl.ANY),
        ],
        out_specs=pl.BlockSpec(memory_space=pltpu.VMEM),
        out_shape=jax.ShapeDtypeStruct((1, H), jnp.float32),
        scratch_shapes=[
            pltpu.VMEM((H, FE), jnp.float32),
            pltpu.VMEM((FE, H), jnp.float32),
            pltpu.SemaphoreType.DMA((CE,)),
            pltpu.SemaphoreType.DMA((CD,)),
        ],
    )(act2, eb2, rb2, router, enc, dec)

    return out.reshape(H)
